# baseline (speedup 1.0000x reference)
import functools

import jax
import jax.numpy as jnp
from jax import lax
from jax.experimental import pallas as pl
from jax.experimental.pallas import tpu as pltpu
from jax.experimental.pallas import tpu_sc as plsc

N = 10000
D = 128
H = 128
L = 64
E = 320000

NPAD = 10240
NCORES = 2
NSUB = 16
NWORKERS = NCORES * NSUB
NHALF = NPAD // 2
UROWS = NHALF + 128
RPT_U = UROWS // NSUB
UREG = 5632
K = 64
UNROLL = 2
NCHUNK = 314
EPW = K * NCHUNK
EPAD = EPW * NSUB
BLK = 512
GRID = NPAD // BLK


def _input_body(x_ref, wx_ref, bx_ref, h_ref):
    h_ref[...] = jnp.tanh(
        jnp.dot(x_ref[...], wx_ref[...], preferred_element_type=jnp.float32)
        + bx_ref[...])


def _tc_input(x, W_x, b_x):
    return pl.pallas_call(
        _input_body,
        grid=(GRID,),
        in_specs=[
            pl.BlockSpec((BLK, D), lambda i: (i, 0)),
            pl.BlockSpec((D, H), lambda i: (0, 0)),
            pl.BlockSpec((1, H), lambda i: (0, 0)),
        ],
        out_specs=pl.BlockSpec((BLK, H), lambda i: (i, 0)),
        out_shape=jax.ShapeDtypeStruct((NPAD, H), jnp.float32),
    )(x, W_x, b_x.reshape(1, H))


def _prep_body(h_ref, ws_ref, wd_ref, g_ref, hr_ref):
    hb = h_ref[...]
    g_ref[:, :H] = jnp.dot(hb, ws_ref[...], preferred_element_type=jnp.float32)
    g_ref[:, H:] = hb
    hr_ref[...] = jnp.dot(hb, wd_ref[...], preferred_element_type=jnp.float32)


def _tc_prep(h, Ws2_i, Wd2_i):
    return pl.pallas_call(
        _prep_body,
        grid=(GRID,),
        in_specs=[
            pl.BlockSpec((BLK, H), lambda i: (i, 0)),
            pl.BlockSpec((H, H), lambda i: (0, 0)),
            pl.BlockSpec((H, H), lambda i: (0, 0)),
        ],
        out_specs=[
            pl.BlockSpec((BLK, 2 * H), lambda i: (i, 0)),
            pl.BlockSpec((BLK, H), lambda i: (i, 0)),
        ],
        out_shape=[
            jax.ShapeDtypeStruct((NPAD, 2 * H), jnp.float32),
            jax.ShapeDtypeStruct((NPAD, H), jnp.float32),
        ],
    )(h, Ws2_i, Wd2_i)


def _u_index_map(i):
    return (i + (i >= NHALF // BLK), 0)


def _agg_from_u(u_ref, s_ref):
    u = u_ref[...]
    s = jnp.sum(s_ref[...], axis=0)[:, None]
    safe = jnp.where(s > 0.0, s, 1.0)
    return jnp.where(s > 0.0, u / safe, 0.0)


def _combine_body(u_ref, s_ref, h_ref, c_ref, wn, bn, wi, bi, wf, bf,
                  wo, bo, wc, bc, h_out, c_out):
    agg = _agg_from_u(u_ref, s_ref)
    ht = jnp.tanh(jnp.dot(agg, wn[...], preferred_element_type=jnp.float32)
                  + bn[...])
    z = jnp.concatenate([h_ref[...], ht], axis=1)
    ig = jax.nn.sigmoid(jnp.dot(z, wi[...], preferred_element_type=jnp.float32) + bi[...])
    fg = jax.nn.sigmoid(jnp.dot(z, wf[...], preferred_element_type=jnp.float32) + bf[...])
    og = jax.nn.sigmoid(jnp.dot(z, wo[...], preferred_element_type=jnp.float32) + bo[...])
    cg = jnp.tanh(jnp.dot(z, wc[...], preferred_element_type=jnp.float32) + bc[...])
    c_new = fg * c_ref[...] + ig * cg
    c_out[...] = c_new
    h_out[...] = og * jnp.tanh(c_new)


def _tc_combine(U, S, h, C, Wn_i, bn_i, Wi_i, bi_i, Wf_i, bf_i, Wo_i, bog_i,
                Wc_i, bc_i):
    wspec = pl.BlockSpec((2 * H, H), lambda i: (0, 0))
    bspec = pl.BlockSpec((1, H), lambda i: (0, 0))
    return pl.pallas_call(
        _combine_body,
        grid=(GRID,),
        in_specs=[
            pl.BlockSpec((BLK, H), _u_index_map),
            pl.BlockSpec((NWORKERS, BLK), lambda i: (0, i)),
            pl.BlockSpec((BLK, H), lambda i: (i, 0)),
            pl.BlockSpec((BLK, H), lambda i: (i, 0)),
            pl.BlockSpec((H, H), lambda i: (0, 0)), bspec,
            wspec, bspec, wspec, bspec, wspec, bspec, wspec, bspec,
        ],
        out_specs=[
            pl.BlockSpec((BLK, H), lambda i: (i, 0)),
            pl.BlockSpec((BLK, H), lambda i: (i, 0)),
        ],
        out_shape=[
            jax.ShapeDtypeStruct((NPAD, H), jnp.float32),
            jax.ShapeDtypeStruct((NPAD, H), jnp.float32),
        ],
    )(U, S, h, C, Wn_i, bn_i.reshape(1, H), Wi_i, bi_i.reshape(1, H),
      Wf_i, bf_i.reshape(1, H), Wo_i, bog_i.reshape(1, H), Wc_i,
      bc_i.reshape(1, H))


def _final_body(u_ref, s_ref, wn_ref, b_ref, out_ref):
    agg = _agg_from_u(u_ref, s_ref)
    out_ref[...] = jnp.dot(agg, wn_ref[...],
                           preferred_element_type=jnp.float32) + b_ref[...]


def _tc_final(U, S, wn_o, b_o):
    return pl.pallas_call(
        _final_body,
        grid=(GRID,),
        in_specs=[
            pl.BlockSpec((BLK, H), _u_index_map),
            pl.BlockSpec((NWORKERS, BLK), lambda i: (0, i)),
            pl.BlockSpec((H, L), lambda i: (0, 0)),
            pl.BlockSpec((1, L), lambda i: (0, 0)),
        ],
        out_specs=pl.BlockSpec((BLK, L), lambda i: (i, 0)),
        out_shape=jax.ShapeDtypeStruct((NPAD, L), jnp.float32),
    )(U, S, wn_o, b_o.reshape(1, L))


_GATHER_DNUMS = lax.GatherDimensionNumbers(
    offset_dims=(), collapsed_slice_dims=(0,), start_index_map=(0,))


def _lane_shuffle(x, idx):
    return lax.gather(x, idx[:, None], _GATHER_DNUMS, slice_sizes=(1,),
                      mode=lax.GatherScatterMode.PROMISE_IN_BOUNDS)


def _hsum16(x):
    lanes = jnp.arange(16, dtype=jnp.int32)
    for sh in (8, 4, 2, 1):
        x = x + _lane_shuffle(x, lanes ^ sh)
    return x


def _edge_body(g_hbm, hr_hbm, src_hbm, dst_hbm, v2_hbm, vs_hbm, u_hbm, s_hbm,
               idx_s0, idx_s1, idx_d0, idx_d1, idx_ds0, idx_ds1, idx_loc,
               g_rows0, g_rows1, hr_rows0, hr_rows1, w_rows, v_v,
               vs_v, s_local, ev_buf, u_sh, sem_g0, sem_g1, sem_h0, sem_h1):
    cid = lax.axis_index("c")
    sid = lax.axis_index("s")
    base = sid * EPW
    lo = cid * NHALF
    zeros16 = jnp.zeros((16,), jnp.float32)
    lanes = jnp.arange(16, dtype=jnp.int32)
    m0 = jnp.where(lanes == 0, 1.0, 0.0)

    def _zero_row(i, _):
        for j in range(H // 16):
            w_rows[i, pl.ds(16 * j, 16)] = zeros16
        return 0
    lax.fori_loop(0, K, _zero_row, 0)

    def _zero_s(i, _):
        s_local[pl.ds(i * 16, 16)] = zeros16
        return 0
    lax.fori_loop(0, (NPAD + 32) // 16, _zero_s, 0)

    def _zero_u(r, _):
        pltpu.sync_copy(w_rows, u_sh.at[pl.ds(sid * RPT_U + r * K, K)])
        return 0
    lax.fori_loop(0, RPT_U // K, _zero_u, 0)
    if RPT_U % K:
        pltpu.sync_copy(
            w_rows.at[pl.ds(0, RPT_U % K)],
            u_sh.at[pl.ds(sid * RPT_U + (RPT_U // K) * K, RPT_U % K)])
    plsc.subcore_barrier()

    pltpu.sync_copy(v2_hbm, v_v)
    pltpu.sync_copy(vs_hbm, vs_v)

    idx_s = (idx_s0, idx_s1)
    idx_d = (idx_d0, idx_d1)
    idx_ds = (idx_ds0, idx_ds1)
    g_rows = (g_rows0, g_rows1)
    hr_rows = (hr_rows0, hr_rows1)
    sem_g = (sem_g0, sem_g1)
    sem_h = (sem_h0, sem_h1)
    vsv = vs_v[pl.ds(0, 16)]

    def _fetch(c, b):
        off = base + c * K
        pltpu.sync_copy(src_hbm.at[pl.ds(off, K)], idx_s[b])
        pltpu.sync_copy(dst_hbm.at[pl.ds(off, K)], idx_d[b])
        pltpu.sync_copy(dst_hbm.at[pl.ds(off, K)], idx_ds[b].at[pl.ds(0, K)])
        pltpu.async_copy(g_hbm.at[idx_s[b]], g_rows[b], sem_g[b])
        pltpu.async_copy(hr_hbm.at[idx_d[b]], hr_rows[b], sem_h[b])

    def _compute(c, b):
        gb = g_rows[b]
        hb = hr_rows[b]

        def _map_idx(q, _):
            dv = idx_d[b][pl.ds(q * 16, 16)]
            lv = dv - lo
            ok = (lv >= 0) & (lv < NHALF)
            trash = NHALF + (dv & 7)
            idx_loc[pl.ds(q * 16, 16)] = jnp.where(ok, lv, trash)
            return 0
        lax.fori_loop(0, K // 16, _map_idx, 0)

        @plsc.parallel_loop(0, K, 1, unroll=UNROLL)
        def _edge(e):
            acc = zeros16
            for j in range(H // 16):
                gl = gb[e, pl.ds(16 * j, 16)]
                hrv = hb[e, pl.ds(16 * j, 16)]
                ex = jnp.exp(gl + hrv)
                acc = acc + v_v[pl.ds(16 * j, 16)] / (ex + 1.0)
            ev = jnp.exp(vsv - _hsum16(acc))
            for j in range(H // 16):
                hsrc = gb[e, pl.ds(H + 16 * j, 16)]
                w_rows[e, pl.ds(16 * j, 16)] = hsrc * ev
            ev_buf[pl.ds(e * 16, 16)] = ev * m0

        def _srmw(e, _):
            d = idx_ds[b][pl.ds(e, 16)][0]
            dl = d - lo
            own = (dl >= 0) & (dl < NHALF)
            si = jnp.where(own, d, NPAD + 16)
            x = s_local[pl.ds(si, 16)]
            s_local[pl.ds(si, 16)] = x + ev_buf[pl.ds(e * 16, 16)]
            return 0
        lax.fori_loop(0, K, _srmw, 0)
        pltpu.sync_copy(w_rows, u_sh.at[idx_loc], add=True)

    _fetch(0, 0)

    def _pair(i, _):
        for b in range(2):
            c = i * 2 + b

            @pl.when(c + 1 < NCHUNK)
            def _():
                _fetch(c + 1, 1 - b)

            pltpu.make_async_copy(g_hbm.at[idx_s[b]], g_rows[b],
                                  sem_g[b]).wait()
            pltpu.make_async_copy(hr_hbm.at[idx_d[b]], hr_rows[b],
                                  sem_h[b]).wait()
            _compute(c, b)
        return 0

    lax.fori_loop(0, NCHUNK // 2, _pair, 0)

    plsc.subcore_barrier()
    pltpu.sync_copy(u_sh.at[pl.ds(sid * RPT_U, RPT_U)],
                    u_hbm.at[pl.ds(cid * UREG + sid * RPT_U, RPT_U)])
    pltpu.sync_copy(s_local.at[pl.ds(0, NPAD)],
                    s_hbm.at[pl.ds((cid * NSUB + sid) * NPAD, NPAD)])


@functools.partial(
    pl.kernel,
    out_type=[
        jax.ShapeDtypeStruct((NCORES * UREG, H), jnp.float32),
        jax.ShapeDtypeStruct((NWORKERS * NPAD,), jnp.float32),
    ],
    mesh=plsc.VectorSubcoreMesh(core_axis_name="c", subcore_axis_name="s"),
    scratch_types=[
        pltpu.VMEM((K,), jnp.int32),
        pltpu.VMEM((K,), jnp.int32),
        pltpu.VMEM((K,), jnp.int32),
        pltpu.VMEM((K,), jnp.int32),
        pltpu.VMEM((K + 16,), jnp.int32),
        pltpu.VMEM((K + 16,), jnp.int32),
        pltpu.VMEM((K,), jnp.int32),
        pltpu.VMEM((K, 2 * H), jnp.float32),
        pltpu.VMEM((K, 2 * H), jnp.float32),
        pltpu.VMEM((K, H), jnp.float32),
        pltpu.VMEM((K, H), jnp.float32),
        pltpu.VMEM((K, H), jnp.float32),
        pltpu.VMEM((H,), jnp.float32),
        pltpu.VMEM((16,), jnp.float32),
        pltpu.VMEM((NPAD + 32,), jnp.float32),
        pltpu.VMEM((K * 16,), jnp.float32),
        pltpu.VMEM_SHARED((UROWS, H), jnp.float32),
        pltpu.SemaphoreType.DMA,
        pltpu.SemaphoreType.DMA,
        pltpu.SemaphoreType.DMA,
        pltpu.SemaphoreType.DMA,
    ],
)
def _sc_edge(g_hbm, hr_hbm, src_hbm, dst_hbm, v2_hbm, vs_hbm, u_hbm, s_hbm,
             idx_s0, idx_s1, idx_d0, idx_d1, idx_ds0, idx_ds1, idx_loc,
             g_rows0, g_rows1, hr_rows0, hr_rows1, w_rows, v_v,
             vs_v, s_local, ev_buf, u_sh, sem_g0, sem_g1, sem_h0, sem_h1):
    _edge_body(g_hbm, hr_hbm, src_hbm, dst_hbm, v2_hbm, vs_hbm, u_hbm, s_hbm,
               idx_s0, idx_s1, idx_d0, idx_d1, idx_ds0, idx_ds1, idx_loc,
               g_rows0, g_rows1, hr_rows0, hr_rows1, w_rows, v_v,
               vs_v, s_local, ev_buf, u_sh, sem_g0, sem_g1, sem_h0, sem_h1)


def kernel(features, edge_index, W_x, b_x, Ws, Wd, v, Wn, bn, Wi, bi, Wf, bf,
           Wo, bog, Wc, bc, ws_o, wd_o, v_o, wn_o, b_o):
    src = edge_index[0].astype(jnp.int32)
    dst = edge_index[1].astype(jnp.int32)
    pad = N + (jnp.arange(EPAD - E, dtype=jnp.int32) % 8)
    srcp = jnp.concatenate([src, pad])
    dstp = jnp.concatenate([dst, pad])

    x = jnp.pad(features, ((0, NPAD - N), (0, 0)))
    h = _tc_input(x, W_x, b_x)
    C = jnp.zeros((NPAD, H), jnp.float32)

    def attn(h, Ws_i, Wd_i, v_i):
        G, HR = _tc_prep(h, 2.0 * Ws_i, 2.0 * Wd_i)
        v2 = 2.0 * v_i
        vs = jnp.full((16,), jnp.sum(v_i), jnp.float32)
        U, S = _sc_edge(G, HR, srcp, dstp, v2, vs)
        return U, S.reshape(NWORKERS, NPAD)

    for i in range(2):
        U, S = attn(h, Ws[i], Wd[i], v[i])
        h, C = _tc_combine(U, S, h, C, Wn[i], bn[i], Wi[i], bi[i], Wf[i],
                           bf[i], Wo[i], bog[i], Wc[i], bc[i])
    U, S = attn(h, ws_o, wd_o, v_o)
    out = _tc_final(U, S, wn_o, b_o)
    return out[:N]


# speedup vs baseline: 1.3669x; 1.3669x over previous
import functools

import jax
import jax.numpy as jnp
from jax import lax
from jax.experimental import pallas as pl
from jax.experimental.pallas import tpu as pltpu
from jax.experimental.pallas import tpu_sc as plsc

N = 10000
D = 128
H = 128
L = 64
E = 320000

NPAD = 10240
NCORES = 2
NSUB = 16
NWORKERS = NCORES * NSUB
NHALF = NPAD // 2
UROWS = NHALF + 128
RPT_U = UROWS // NSUB
UREG = 5632
K = 64
UNROLL = 1
NCHUNK = 314
EPW = K * NCHUNK
EPAD = EPW * NSUB
PKW = 2 * K + 16
BLK = 512
GRID = NPAD // BLK


def _input_body(x_ref, wx_ref, bx_ref, h_ref):
    h_ref[...] = jnp.tanh(
        jnp.dot(x_ref[...], wx_ref[...], preferred_element_type=jnp.float32)
        + bx_ref[...])


def _tc_input(x, W_x, b_x):
    return pl.pallas_call(
        _input_body,
        grid=(GRID,),
        in_specs=[
            pl.BlockSpec((BLK, D), lambda i: (i, 0)),
            pl.BlockSpec((D, H), lambda i: (0, 0)),
            pl.BlockSpec((1, H), lambda i: (0, 0)),
        ],
        out_specs=pl.BlockSpec((BLK, H), lambda i: (i, 0)),
        out_shape=jax.ShapeDtypeStruct((NPAD, H), jnp.float32),
    )(x, W_x, b_x.reshape(1, H))


def _prep_body(h_ref, ws_ref, wd_ref, g_ref, hr_ref):
    hb = h_ref[...]
    g_ref[:, :H] = jnp.dot(hb, ws_ref[...], preferred_element_type=jnp.float32)
    g_ref[:, H:] = hb
    hr_ref[...] = jnp.dot(hb, wd_ref[...], preferred_element_type=jnp.float32)


def _tc_prep(h, Ws2_i, Wd2_i):
    return pl.pallas_call(
        _prep_body,
        grid=(GRID,),
        in_specs=[
            pl.BlockSpec((BLK, H), lambda i: (i, 0)),
            pl.BlockSpec((H, H), lambda i: (0, 0)),
            pl.BlockSpec((H, H), lambda i: (0, 0)),
        ],
        out_specs=[
            pl.BlockSpec((BLK, 2 * H), lambda i: (i, 0)),
            pl.BlockSpec((BLK, H), lambda i: (i, 0)),
        ],
        out_shape=[
            jax.ShapeDtypeStruct((NPAD, 2 * H), jnp.float32),
            jax.ShapeDtypeStruct((NPAD, H), jnp.float32),
        ],
    )(h, Ws2_i, Wd2_i)


def _u_index_map(i):
    return (i + (i >= NHALF // BLK), 0)


def _agg_from_u(u_ref, s_ref):
    u = u_ref[...]
    s = jnp.sum(s_ref[...], axis=0)[:, None]
    safe = jnp.where(s > 0.0, s, 1.0)
    return jnp.where(s > 0.0, u / safe, 0.0)


def _combine_body(u_ref, s_ref, h_ref, c_ref, wn, bn, wi, bi, wf, bf,
                  wo, bo, wc, bc, h_out, c_out):
    agg = _agg_from_u(u_ref, s_ref)
    ht = jnp.tanh(jnp.dot(agg, wn[...], preferred_element_type=jnp.float32)
                  + bn[...])
    z = jnp.concatenate([h_ref[...], ht], axis=1)
    ig = jax.nn.sigmoid(jnp.dot(z, wi[...], preferred_element_type=jnp.float32) + bi[...])
    fg = jax.nn.sigmoid(jnp.dot(z, wf[...], preferred_element_type=jnp.float32) + bf[...])
    og = jax.nn.sigmoid(jnp.dot(z, wo[...], preferred_element_type=jnp.float32) + bo[...])
    cg = jnp.tanh(jnp.dot(z, wc[...], preferred_element_type=jnp.float32) + bc[...])
    c_new = fg * c_ref[...] + ig * cg
    c_out[...] = c_new
    h_out[...] = og * jnp.tanh(c_new)


def _tc_combine(U, S, h, C, Wn_i, bn_i, Wi_i, bi_i, Wf_i, bf_i, Wo_i, bog_i,
                Wc_i, bc_i):
    wspec = pl.BlockSpec((2 * H, H), lambda i: (0, 0))
    bspec = pl.BlockSpec((1, H), lambda i: (0, 0))
    return pl.pallas_call(
        _combine_body,
        grid=(GRID,),
        in_specs=[
            pl.BlockSpec((BLK, H), _u_index_map),
            pl.BlockSpec((NWORKERS, BLK), lambda i: (0, i)),
            pl.BlockSpec((BLK, H), lambda i: (i, 0)),
            pl.BlockSpec((BLK, H), lambda i: (i, 0)),
            pl.BlockSpec((H, H), lambda i: (0, 0)), bspec,
            wspec, bspec, wspec, bspec, wspec, bspec, wspec, bspec,
        ],
        out_specs=[
            pl.BlockSpec((BLK, H), lambda i: (i, 0)),
            pl.BlockSpec((BLK, H), lambda i: (i, 0)),
        ],
        out_shape=[
            jax.ShapeDtypeStruct((NPAD, H), jnp.float32),
            jax.ShapeDtypeStruct((NPAD, H), jnp.float32),
        ],
    )(U, S, h, C, Wn_i, bn_i.reshape(1, H), Wi_i, bi_i.reshape(1, H),
      Wf_i, bf_i.reshape(1, H), Wo_i, bog_i.reshape(1, H), Wc_i,
      bc_i.reshape(1, H))


def _final_body(u_ref, s_ref, wn_ref, b_ref, out_ref):
    agg = _agg_from_u(u_ref, s_ref)
    out_ref[...] = jnp.dot(agg, wn_ref[...],
                           preferred_element_type=jnp.float32) + b_ref[...]


def _tc_final(U, S, wn_o, b_o):
    return pl.pallas_call(
        _final_body,
        grid=(GRID,),
        in_specs=[
            pl.BlockSpec((BLK, H), _u_index_map),
            pl.BlockSpec((NWORKERS, BLK), lambda i: (0, i)),
            pl.BlockSpec((H, L), lambda i: (0, 0)),
            pl.BlockSpec((1, L), lambda i: (0, 0)),
        ],
        out_specs=pl.BlockSpec((BLK, L), lambda i: (i, 0)),
        out_shape=jax.ShapeDtypeStruct((NPAD, L), jnp.float32),
    )(U, S, wn_o, b_o.reshape(1, L))


_GATHER_DNUMS = lax.GatherDimensionNumbers(
    offset_dims=(), collapsed_slice_dims=(0,), start_index_map=(0,))


def _lane_shuffle(x, idx):
    return lax.gather(x, idx[:, None], _GATHER_DNUMS, slice_sizes=(1,),
                      mode=lax.GatherScatterMode.PROMISE_IN_BOUNDS)


def _hsum16(x):
    lanes = jnp.arange(16, dtype=jnp.int32)
    for sh in (8, 4, 2, 1):
        x = x + _lane_shuffle(x, lanes ^ sh)
    return x


def _edge_body(g_hbm, hr_hbm, epk_hbm, v2_hbm, vs_hbm, u_hbm, s_hbm,
               idx_pk0, idx_pk1, idx_loc0, idx_loc1,
               g_rows0, g_rows1, hr_rows0, hr_rows1, w_rows0, w_rows1, v_v,
               vs_v, s_local, ev_buf, u_sh,
               sem_g0, sem_g1, sem_h0, sem_h1, sem_w0, sem_w1):
    cid = lax.axis_index("c")
    sid = lax.axis_index("s")
    base = sid * EPW
    lo = cid * NHALF
    zeros16 = jnp.zeros((16,), jnp.float32)
    lanes = jnp.arange(16, dtype=jnp.int32)
    m0 = jnp.where(lanes == 0, 1.0, 0.0)

    idx_pk = (idx_pk0, idx_pk1)
    idx_loc = (idx_loc0, idx_loc1)
    g_rows = (g_rows0, g_rows1)
    hr_rows = (hr_rows0, hr_rows1)
    w_rows = (w_rows0, w_rows1)
    sem_g = (sem_g0, sem_g1)
    sem_h = (sem_h0, sem_h1)
    sem_w = (sem_w0, sem_w1)

    def _zero_row(i, _):
        for j in range(H // 16):
            w_rows0[i, pl.ds(16 * j, 16)] = zeros16
        return 0
    lax.fori_loop(0, K, _zero_row, 0)

    def _zero_s(i, _):
        s_local[pl.ds(i * 16, 16)] = zeros16
        return 0
    lax.fori_loop(0, (NPAD + 32) // 16, _zero_s, 0)

    def _zero_u(r, _):
        pltpu.sync_copy(w_rows0, u_sh.at[pl.ds(sid * RPT_U + r * K, K)])
        return 0
    lax.fori_loop(0, RPT_U // K, _zero_u, 0)
    if RPT_U % K:
        pltpu.sync_copy(
            w_rows0.at[pl.ds(0, RPT_U % K)],
            u_sh.at[pl.ds(sid * RPT_U + (RPT_U // K) * K, RPT_U % K)])
    plsc.subcore_barrier()

    pltpu.sync_copy(v2_hbm, v_v)
    pltpu.sync_copy(vs_hbm, vs_v)
    vsv = vs_v[pl.ds(0, 16)]

    def _fetch(c, b):
        off = (sid * NCHUNK + c) * PKW
        pltpu.sync_copy(epk_hbm.at[pl.ds(off, PKW)], idx_pk[b])
        pltpu.async_copy(g_hbm.at[idx_pk[b].at[pl.ds(0, K)]],
                         g_rows[b], sem_g[b])
        pltpu.async_copy(hr_hbm.at[idx_pk[b].at[pl.ds(K, K)]],
                         hr_rows[b], sem_h[b])

    def _compute(c, b):
        gb = g_rows[b]
        hb = hr_rows[b]
        wb = w_rows[b]

        @pl.when(c >= 2)
        def _():
            pltpu.make_async_copy(wb, u_sh.at[idx_loc[b]], sem_w[b]).wait()

        def _map_idx(q, _):
            dv = idx_pk[b][pl.ds(K + q * 16, 16)]
            lv = dv - lo
            ok = (lv >= 0) & (lv < NHALF)
            trash = NHALF + (dv & 7)
            idx_loc[b][pl.ds(q * 16, 16)] = jnp.where(ok, lv, trash)
            return 0
        lax.fori_loop(0, K // 16, _map_idx, 0)

        @plsc.parallel_loop(0, K, 1, unroll=UNROLL)
        def _edge(e):
            acc = zeros16
            for j in range(H // 16):
                gl = gb[e, pl.ds(16 * j, 16)]
                hrv = hb[e, pl.ds(16 * j, 16)]
                ex = jnp.exp(gl + hrv)
                acc = acc + v_v[pl.ds(16 * j, 16)] / (ex + 1.0)
            ev = jnp.exp(vsv - _hsum16(acc))
            for j in range(H // 16):
                hsrc = gb[e, pl.ds(H + 16 * j, 16)]
                wb[e, pl.ds(16 * j, 16)] = hsrc * ev
            ev_buf[pl.ds(e * 16, 16)] = ev * m0

        def _srmw(e, _):
            d = idx_pk[b][pl.ds(K + e, 16)][0]
            dl = d - lo
            own = (dl >= 0) & (dl < NHALF)
            si = jnp.where(own, d, NPAD + 16)
            x = s_local[pl.ds(si, 16)]
            s_local[pl.ds(si, 16)] = x + ev_buf[pl.ds(e * 16, 16)]
            return 0
        lax.fori_loop(0, K, _srmw, 0)
        pltpu.async_copy(wb, u_sh.at[idx_loc[b]], sem_w[b], add=True)

    _fetch(0, 0)

    def _pair(i, _):
        for b in range(2):
            c = i * 2 + b

            @pl.when(c + 1 < NCHUNK)
            def _():
                _fetch(c + 1, 1 - b)

            pltpu.make_async_copy(g_hbm.at[idx_pk[b].at[pl.ds(0, K)]],
                                  g_rows[b], sem_g[b]).wait()
            pltpu.make_async_copy(hr_hbm.at[idx_pk[b].at[pl.ds(K, K)]],
                                  hr_rows[b], sem_h[b]).wait()
            _compute(c, b)
        return 0

    lax.fori_loop(0, NCHUNK // 2, _pair, 0)

    for b in range(2):
        pltpu.make_async_copy(w_rows[b], u_sh.at[idx_loc[b]],
                              sem_w[b]).wait()

    plsc.subcore_barrier()
    pltpu.sync_copy(u_sh.at[pl.ds(sid * RPT_U, RPT_U)],
                    u_hbm.at[pl.ds(cid * UREG + sid * RPT_U, RPT_U)])
    pltpu.sync_copy(s_local.at[pl.ds(0, NPAD)],
                    s_hbm.at[pl.ds((cid * NSUB + sid) * NPAD, NPAD)])


@functools.partial(
    pl.kernel,
    out_type=[
        jax.ShapeDtypeStruct((NCORES * UREG, H), jnp.float32),
        jax.ShapeDtypeStruct((NWORKERS * NPAD,), jnp.float32),
    ],
    mesh=plsc.VectorSubcoreMesh(core_axis_name="c", subcore_axis_name="s"),
    scratch_types=[
        pltpu.VMEM((PKW,), jnp.int32),
        pltpu.VMEM((PKW,), jnp.int32),
        pltpu.VMEM((K,), jnp.int32),
        pltpu.VMEM((K,), jnp.int32),
        pltpu.VMEM((K, 2 * H), jnp.float32),
        pltpu.VMEM((K, 2 * H), jnp.float32),
        pltpu.VMEM((K, H), jnp.float32),
        pltpu.VMEM((K, H), jnp.float32),
        pltpu.VMEM((K, H), jnp.float32),
        pltpu.VMEM((K, H), jnp.float32),
        pltpu.VMEM((H,), jnp.float32),
        pltpu.VMEM((16,), jnp.float32),
        pltpu.VMEM((NPAD + 32,), jnp.float32),
        pltpu.VMEM((K * 16,), jnp.float32),
        pltpu.VMEM_SHARED((UROWS, H), jnp.float32),
        pltpu.SemaphoreType.DMA,
        pltpu.SemaphoreType.DMA,
        pltpu.SemaphoreType.DMA,
        pltpu.SemaphoreType.DMA,
        pltpu.SemaphoreType.DMA,
        pltpu.SemaphoreType.DMA,
    ],
)
def _sc_edge(g_hbm, hr_hbm, epk_hbm, v2_hbm, vs_hbm, u_hbm, s_hbm,
             idx_pk0, idx_pk1, idx_loc0, idx_loc1,
             g_rows0, g_rows1, hr_rows0, hr_rows1, w_rows0, w_rows1, v_v,
             vs_v, s_local, ev_buf, u_sh,
             sem_g0, sem_g1, sem_h0, sem_h1, sem_w0, sem_w1):
    _edge_body(g_hbm, hr_hbm, epk_hbm, v2_hbm, vs_hbm, u_hbm, s_hbm,
               idx_pk0, idx_pk1, idx_loc0, idx_loc1,
               g_rows0, g_rows1, hr_rows0, hr_rows1, w_rows0, w_rows1, v_v,
               vs_v, s_local, ev_buf, u_sh,
               sem_g0, sem_g1, sem_h0, sem_h1, sem_w0, sem_w1)


def kernel(features, edge_index, W_x, b_x, Ws, Wd, v, Wn, bn, Wi, bi, Wf, bf,
           Wo, bog, Wc, bc, ws_o, wd_o, v_o, wn_o, b_o):
    src = edge_index[0].astype(jnp.int32)
    dst = edge_index[1].astype(jnp.int32)
    pad = N + (jnp.arange(EPAD + K - E, dtype=jnp.int32) % 8)
    srcp = jnp.concatenate([src, pad])
    dstp = jnp.concatenate([dst, pad])
    wc = NSUB * NCHUNK
    src_r = srcp[:EPAD].reshape(wc, K)
    dst_r = dstp[:EPAD].reshape(wc, K)
    tails = dstp[K:EPAD + K].reshape(wc, K)[:, :16]
    epk = jnp.concatenate([src_r, dst_r, tails], axis=1).reshape(-1)

    x = jnp.pad(features, ((0, NPAD - N), (0, 0)))
    h = _tc_input(x, W_x, b_x)
    C = jnp.zeros((NPAD, H), jnp.float32)

    def attn(h, Ws_i, Wd_i, v_i):
        G, HR = _tc_prep(h, 2.0 * Ws_i, 2.0 * Wd_i)
        v2 = 2.0 * v_i
        vs = jnp.full((16,), jnp.sum(v_i), jnp.float32)
        U, S = _sc_edge(G, HR, epk, v2, vs)
        return U, S.reshape(NWORKERS, NPAD)

    for i in range(2):
        U, S = attn(h, Ws[i], Wd[i], v[i])
        h, C = _tc_combine(U, S, h, C, Wn[i], bn[i], Wi[i], bi[i], Wf[i],
                           bf[i], Wo[i], bog[i], Wc[i], bc[i])
    U, S = attn(h, ws_o, wd_o, v_o)
    out = _tc_final(U, S, wn_o, b_o)
    return out[:N]
